# baseline (speedup 1.0000x reference)
import functools

import jax
import jax.numpy as jnp
from jax import lax
from jax.experimental import pallas as pl
from jax.experimental.pallas import tpu as pltpu
from jax.experimental.pallas import tpu_sc as plsc

B, S, I, H, MD, M, O, TOPK = 32, 16, 1024, 1024, 256, 1024, 1024, 3
N = B * S
G4 = 4 * H
NEG = -1e30


def _fwd_body(xt_ref, wih_ref, whh_ref, bih_ref, bhh_ref, wq_ref, bq_ref,
              keys_ref, vals_ref, wa_ref, ba_ref, ctrl_ref, sims_ref, va_ref,
              xg_ref, hs_ref, h_ref, c_ref):
    xg_ref[...] = (
        lax.dot_general(xt_ref[...], wih_ref[...], (((1,), (1,)), ((), ())))
        + bih_ref[...] + bhh_ref[...]
    )
    h_ref[...] = jnp.zeros_like(h_ref)
    c_ref[...] = jnp.zeros_like(c_ref)

    def step(t, carry):
        g = xg_ref[pl.ds(t * B, B), :] + lax.dot_general(
            h_ref[...], whh_ref[...], (((1,), (1,)), ((), ()))
        )
        i_g = jax.nn.sigmoid(g[:, 0 * H:1 * H])
        f_g = jax.nn.sigmoid(g[:, 1 * H:2 * H])
        g_g = jnp.tanh(g[:, 2 * H:3 * H])
        o_g = jax.nn.sigmoid(g[:, 3 * H:4 * H])
        c = f_g * c_ref[...] + i_g * g_g
        h = o_g * jnp.tanh(c)
        c_ref[...] = c
        h_ref[...] = h
        hs_ref[pl.ds(t * B, B), :] = h
        return carry

    lax.fori_loop(0, S, step, 0)

    ctrl = hs_ref[...]
    ctrl_ref[...] = ctrl
    q = (
        lax.dot_general(ctrl, wq_ref[...], (((1,), (1,)), ((), ())))
        + bq_ref[...]
    )
    qn = q / jnp.maximum(
        jnp.sqrt(jnp.sum(q * q, axis=1, keepdims=True)), 1e-12
    )
    k = keys_ref[...]
    kn = k / jnp.maximum(
        jnp.sqrt(jnp.sum(k * k, axis=1, keepdims=True)), 1e-12
    )
    sims_ref[...] = jnp.reshape(
        lax.dot_general(qn, kn, (((1,), (1,)), ((), ()))), (N * M,)
    )
    va_ref[...] = jnp.sum(vals_ref[...] * wa_ref[...], axis=1) + ba_ref[0, 0]


def _forward_tc(xt, W_ih, W_hh, b_ih, b_hh, Wq, bq, mem_keys, mem_values,
                Wa, ba):
    return pl.pallas_call(
        _fwd_body,
        out_shape=(
            jax.ShapeDtypeStruct((N, H), jnp.float32),
            jax.ShapeDtypeStruct((N * M,), jnp.float32),
            jax.ShapeDtypeStruct((M,), jnp.float32),
        ),
        scratch_shapes=[
            pltpu.VMEM((N, G4), jnp.float32),
            pltpu.VMEM((N, H), jnp.float32),
            pltpu.VMEM((B, H), jnp.float32),
            pltpu.VMEM((B, H), jnp.float32),
        ],
    )(xt, W_ih, W_hh, b_ih, b_hh, Wq, bq, mem_keys, mem_values, Wa, ba)


_NW = 32
_RW = N // _NW


@functools.lru_cache(maxsize=1)
def _make_retrieve():
    mesh = plsc.VectorSubcoreMesh(core_axis_name="c", subcore_axis_name="s")

    @functools.partial(
        pl.kernel,
        out_type=jax.ShapeDtypeStruct((N, MD), jnp.float32),
        mesh=mesh,
        compiler_params=pltpu.CompilerParams(needs_layout_passes=False),
        scratch_types=[
            pltpu.VMEM((_RW * M,), jnp.float32),
            pltpu.VMEM((M,), jnp.float32),
            pltpu.VMEM((_RW * TOPK,), jnp.int32),
            pltpu.VMEM((_RW * TOPK, MD), jnp.float32),
            pltpu.VMEM((_RW, MD), jnp.float32),
            pltpu.VMEM((TOPK * 16,), jnp.float32),
            pltpu.SemaphoreType.DMA,
        ],
    )
    def retrieve(sims_hbm, va_hbm, vals_hbm, out_hbm,
                 sims_v, va_v, idx_v, rows_v, out_v, att_v, sem):
        wid = lax.axis_index("s") * 2 + lax.axis_index("c")
        base = wid * _RW
        pltpu.sync_copy(sims_hbm.at[pl.ds(base * M, _RW * M)], sims_v)
        pltpu.sync_copy(va_hbm, va_v)

        lane = lax.broadcasted_iota(jnp.int32, (16,), 0)
        zero = jnp.zeros((16,), jnp.int32)
        negs = jnp.full((16,), NEG, jnp.float32)
        big = jnp.full((16,), 2 ** 30, jnp.int32)

        def topk_row(r, carry):
            def chunk(c, st):
                bv1, bv2, bv3, bi1, bi2, bi3, gi = st
                v = sims_v[pl.ds(r * M + c * 16, 16)]
                c1 = v > bv1
                c2 = v > bv2
                c3 = v > bv3
                nb1 = jnp.where(c1, v, bv1)
                ni1 = jnp.where(c1, gi, bi1)
                nb2 = jnp.where(c1, bv1, jnp.where(c2, v, bv2))
                ni2 = jnp.where(c1, bi1, jnp.where(c2, gi, bi2))
                nb3 = jnp.where(c2, bv2, jnp.where(c3, v, bv3))
                ni3 = jnp.where(c2, bi2, jnp.where(c3, gi, bi3))
                return nb1, nb2, nb3, ni1, ni2, ni3, gi + 16

            bv1, bv2, bv3, bi1, bi2, bi3, _ = lax.fori_loop(
                0, M // 16, chunk,
                (negs, negs, negs, zero, zero, zero, lane), unroll=8,
            )

            def extract(st):
                bv1, bv2, bv3, bi1, bi2, bi3 = st
                m = jnp.max(bv1)
                i_t = jnp.min(jnp.where(bv1 == m, bi1, big))
                shift = bi1 == i_t
                bv1 = jnp.where(shift, bv2, bv1)
                bi1 = jnp.where(shift, bi2, bi1)
                bv2 = jnp.where(shift, bv3, bv2)
                bi2 = jnp.where(shift, bi3, bi2)
                return (bv1, bv2, bv3, bi1, bi2, bi3), i_t

            st = (bv1, bv2, bv3, bi1, bi2, bi3)
            st, i0 = extract(st)
            st, i1 = extract(st)
            _, i2 = extract(st)
            vec = jnp.where(lane == 0, zero + i0,
                            jnp.where(lane == 1, zero + i1, zero + i2))
            plsc.store_scatter(idx_v, [r * TOPK + lane], vec,
                               mask=lane < TOPK)
            return carry

        lax.fori_loop(0, _RW, topk_row, 0)

        i0 = plsc.load_gather(idx_v, [lane * TOPK])
        i1 = plsc.load_gather(idx_v, [lane * TOPK + 1])
        i2 = plsc.load_gather(idx_v, [lane * TOPK + 2])
        l0 = plsc.load_gather(va_v, [i0])
        l1 = plsc.load_gather(va_v, [i1])
        l2 = plsc.load_gather(va_v, [i2])
        m = jnp.maximum(jnp.maximum(l0, l1), l2)
        e0 = jnp.exp(l0 - m)
        e1 = jnp.exp(l1 - m)
        e2 = jnp.exp(l2 - m)
        inv = 1.0 / (e0 + e1 + e2)
        att_v[pl.ds(0, 16)] = e0 * inv
        att_v[pl.ds(16, 16)] = e1 * inv
        att_v[pl.ds(32, 16)] = e2 * inv

        pltpu.async_copy(vals_hbm.at[idx_v], rows_v, sem).wait()

        def combine_row(r, carry):
            a0 = plsc.load_gather(att_v, [zero + r])
            a1 = plsc.load_gather(att_v, [zero + (16 + r)])
            a2 = plsc.load_gather(att_v, [zero + (32 + r)])

            def cchunk(j, carry2):
                sl = pl.ds(j * 16, 16)
                out_v[r, sl] = (
                    a0 * rows_v[r * 3 + 0, sl]
                    + a1 * rows_v[r * 3 + 1, sl]
                    + a2 * rows_v[r * 3 + 2, sl]
                )
                return carry2

            lax.fori_loop(0, MD // 16, cchunk, 0, unroll=4)
            return carry

        lax.fori_loop(0, _RW, combine_row, 0)
        pltpu.sync_copy(out_v, out_hbm.at[pl.ds(base, _RW)])

    return retrieve


def _outa_body(ctrl_ref, woh_ref, bo_ref, out_ref):
    out_ref[...] = (
        lax.dot_general(ctrl_ref[...], woh_ref[...], (((1,), (1,)), ((), ())))
        + bo_ref[...]
    )


def _outa(ctrl, Wo, bo):
    return pl.pallas_call(
        _outa_body,
        grid=(1,),
        in_specs=[
            pl.BlockSpec((N, H), lambda i: (0, 0)),
            pl.BlockSpec((O, H), lambda i: (0, 0)),
            pl.BlockSpec((1, O), lambda i: (0, 0)),
        ],
        out_specs=pl.BlockSpec((N, O), lambda i: (0, 0)),
        out_shape=jax.ShapeDtypeStruct((N, O), jnp.float32),
    )(ctrl, Wo, bo)


def _outb_body(outa_ref, mem_ref, wc_ref, bc_ref, wom_ref, out_ref):
    mem_out = (
        lax.dot_general(mem_ref[...], wc_ref[...], (((1,), (1,)), ((), ())))
        + bc_ref[...]
    )
    out_ref[...] = outa_ref[...] + lax.dot_general(
        mem_out, wom_ref[...], (((1,), (1,)), ((), ()))
    )


def _outb(outa, mem_raw, Wc, bc, Wo):
    return pl.pallas_call(
        _outb_body,
        grid=(1,),
        in_specs=[
            pl.BlockSpec((N, O), lambda s: (0, 0)),
            pl.BlockSpec((N, MD), lambda s: (0, 0)),
            pl.BlockSpec((MD, MD), lambda s: (0, 0)),
            pl.BlockSpec((1, MD), lambda s: (0, 0)),
            pl.BlockSpec((O, MD), lambda s: (0, H // MD)),
        ],
        out_specs=pl.BlockSpec((N, O), lambda s: (0, 0)),
        out_shape=jax.ShapeDtypeStruct((N, O), jnp.float32),
    )(outa, mem_raw, Wc, bc, Wo)


def kernel(x, W_ih, W_hh, b_ih, b_hh, Wq, bq, Wa, ba, Wc, bc, Wo, bo,
           mem_keys, mem_values):
    xt = jnp.transpose(x, (1, 0, 2)).reshape(N, I)
    ctrl, sims, va2 = _forward_tc(xt, W_ih, W_hh, b_ih.reshape(1, G4),
                                  b_hh.reshape(1, G4), Wq,
                                  bq.reshape(1, MD), mem_keys, mem_values,
                                  Wa.reshape(1, MD), ba.reshape(1, 1))
    mem_raw = _make_retrieve()(sims, va2, mem_values)
    outa = _outa(ctrl, Wo, bo.reshape(1, O))
    out = _outb(outa, mem_raw, Wc, bc.reshape(1, MD), Wo)
    return out.reshape(S, B, O).transpose(1, 0, 2)


# speedup vs baseline: 1.0324x; 1.0324x over previous
import functools

import jax
import jax.numpy as jnp
from jax import lax
from jax.experimental import pallas as pl
from jax.experimental.pallas import tpu as pltpu
from jax.experimental.pallas import tpu_sc as plsc

B, S, I, H, MD, M, O, TOPK = 32, 16, 1024, 1024, 256, 1024, 1024, 3
N = B * S
G4 = 4 * H
NEG = -1e30


def _fwd_body(xt_ref, wih_ref, whh_ref, bih_ref, bhh_ref, wq_ref, bq_ref,
              keys_ref, vals_ref, wa_ref, ba_ref, ctrl_ref, sims_ref, va_ref,
              xg_ref, hs_ref, h_ref, c_ref):
    xg_ref[...] = (
        lax.dot_general(xt_ref[...], wih_ref[...], (((1,), (1,)), ((), ())))
        + bih_ref[...] + bhh_ref[...]
    )
    h_ref[...] = jnp.zeros_like(h_ref)
    c_ref[...] = jnp.zeros_like(c_ref)

    def step(t, carry):
        g = xg_ref[pl.ds(t * B, B), :] + lax.dot_general(
            h_ref[...], whh_ref[...], (((1,), (1,)), ((), ()))
        )
        i_g = jax.nn.sigmoid(g[:, 0 * H:1 * H])
        f_g = jax.nn.sigmoid(g[:, 1 * H:2 * H])
        g_g = jnp.tanh(g[:, 2 * H:3 * H])
        o_g = jax.nn.sigmoid(g[:, 3 * H:4 * H])
        c = f_g * c_ref[...] + i_g * g_g
        h = o_g * jnp.tanh(c)
        c_ref[...] = c
        h_ref[...] = h
        hs_ref[pl.ds(t * B, B), :] = h
        return carry

    lax.fori_loop(0, S, step, 0)

    ctrl = hs_ref[...]
    ctrl_ref[...] = ctrl
    q = (
        lax.dot_general(ctrl, wq_ref[...], (((1,), (1,)), ((), ())))
        + bq_ref[...]
    )
    qn = q / jnp.maximum(
        jnp.sqrt(jnp.sum(q * q, axis=1, keepdims=True)), 1e-12
    )
    k = keys_ref[...]
    kn = k / jnp.maximum(
        jnp.sqrt(jnp.sum(k * k, axis=1, keepdims=True)), 1e-12
    )
    sims_ref[...] = jnp.reshape(
        lax.dot_general(qn, kn, (((1,), (1,)), ((), ()))), (N * M,)
    )
    va_ref[...] = jnp.sum(vals_ref[...] * wa_ref[...], axis=1) + ba_ref[0, 0]


def _forward_tc(xt, W_ih, W_hh, b_ih, b_hh, Wq, bq, mem_keys, mem_values,
                Wa, ba):
    return pl.pallas_call(
        _fwd_body,
        out_shape=(
            jax.ShapeDtypeStruct((N, H), jnp.float32),
            jax.ShapeDtypeStruct((N * M,), jnp.float32),
            jax.ShapeDtypeStruct((M,), jnp.float32),
        ),
        scratch_shapes=[
            pltpu.VMEM((N, G4), jnp.float32),
            pltpu.VMEM((N, H), jnp.float32),
            pltpu.VMEM((B, H), jnp.float32),
            pltpu.VMEM((B, H), jnp.float32),
        ],
    )(xt, W_ih, W_hh, b_ih, b_hh, Wq, bq, mem_keys, mem_values, Wa, ba)


_NW = 32
_RW = N // _NW


@functools.lru_cache(maxsize=1)
def _make_retrieve():
    mesh = plsc.VectorSubcoreMesh(core_axis_name="c", subcore_axis_name="s")

    @functools.partial(
        pl.kernel,
        out_type=jax.ShapeDtypeStruct((N, MD), jnp.float32),
        mesh=mesh,
        compiler_params=pltpu.CompilerParams(needs_layout_passes=False),
        scratch_types=[
            pltpu.VMEM((_RW * M,), jnp.float32),
            pltpu.VMEM((M,), jnp.float32),
            pltpu.VMEM((_RW * TOPK,), jnp.int32),
            pltpu.VMEM((_RW * TOPK, MD), jnp.float32),
            pltpu.VMEM((_RW, MD), jnp.float32),
            pltpu.VMEM((TOPK * 16,), jnp.float32),
            pltpu.SemaphoreType.DMA,
        ],
    )
    def retrieve(sims_hbm, va_hbm, vals_hbm, out_hbm,
                 sims_v, va_v, idx_v, rows_v, out_v, att_v, sem):
        wid = lax.axis_index("s") * 2 + lax.axis_index("c")
        base = wid * _RW
        pltpu.sync_copy(sims_hbm.at[pl.ds(base * M, _RW * M)], sims_v)
        pltpu.sync_copy(va_hbm, va_v)

        lane = lax.broadcasted_iota(jnp.int32, (16,), 0)
        zero = jnp.zeros((16,), jnp.int32)
        negs = jnp.full((16,), NEG, jnp.float32)
        big = jnp.full((16,), 2 ** 30, jnp.int32)

        def topk_row(r, carry):
            def chunk(c, st):
                bv1, bv2, bv3, bi1, bi2, bi3, gi = st
                v = sims_v[pl.ds(r * M + c * 16, 16)]
                c1 = v > bv1
                c2 = v > bv2
                c3 = v > bv3
                nb1 = jnp.where(c1, v, bv1)
                ni1 = jnp.where(c1, gi, bi1)
                nb2 = jnp.where(c1, bv1, jnp.where(c2, v, bv2))
                ni2 = jnp.where(c1, bi1, jnp.where(c2, gi, bi2))
                nb3 = jnp.where(c2, bv2, jnp.where(c3, v, bv3))
                ni3 = jnp.where(c2, bi2, jnp.where(c3, gi, bi3))
                return nb1, nb2, nb3, ni1, ni2, ni3, gi + 16

            bv1, bv2, bv3, bi1, bi2, bi3, _ = lax.fori_loop(
                0, M // 16, chunk,
                (negs, negs, negs, zero, zero, zero, lane), unroll=8,
            )

            def extract(st):
                bv1, bv2, bv3, bi1, bi2, bi3 = st
                m = jnp.max(bv1)
                i_t = jnp.min(jnp.where(bv1 == m, bi1, big))
                shift = bi1 == i_t
                bv1 = jnp.where(shift, bv2, bv1)
                bi1 = jnp.where(shift, bi2, bi1)
                bv2 = jnp.where(shift, bv3, bv2)
                bi2 = jnp.where(shift, bi3, bi2)
                return (bv1, bv2, bv3, bi1, bi2, bi3), i_t

            st = (bv1, bv2, bv3, bi1, bi2, bi3)
            st, i0 = extract(st)
            st, i1 = extract(st)
            _, i2 = extract(st)
            vec = jnp.where(lane == 0, zero + i0,
                            jnp.where(lane == 1, zero + i1, zero + i2))
            plsc.store_scatter(idx_v, [r * TOPK + lane], vec,
                               mask=lane < TOPK)
            return carry

        lax.fori_loop(0, _RW, topk_row, 0)

        i0 = plsc.load_gather(idx_v, [lane * TOPK])
        i1 = plsc.load_gather(idx_v, [lane * TOPK + 1])
        i2 = plsc.load_gather(idx_v, [lane * TOPK + 2])
        l0 = plsc.load_gather(va_v, [i0])
        l1 = plsc.load_gather(va_v, [i1])
        l2 = plsc.load_gather(va_v, [i2])
        m = jnp.maximum(jnp.maximum(l0, l1), l2)
        e0 = jnp.exp(l0 - m)
        e1 = jnp.exp(l1 - m)
        e2 = jnp.exp(l2 - m)
        inv = 1.0 / (e0 + e1 + e2)
        att_v[pl.ds(0, 16)] = e0 * inv
        att_v[pl.ds(16, 16)] = e1 * inv
        att_v[pl.ds(32, 16)] = e2 * inv

        pltpu.async_copy(vals_hbm.at[idx_v], rows_v, sem).wait()

        def combine_row(r, carry):
            a0 = plsc.load_gather(att_v, [zero + r])
            a1 = plsc.load_gather(att_v, [zero + (16 + r)])
            a2 = plsc.load_gather(att_v, [zero + (32 + r)])

            def cchunk(j, carry2):
                sl = pl.ds(j * 16, 16)
                out_v[r, sl] = (
                    a0 * rows_v[r * 3 + 0, sl]
                    + a1 * rows_v[r * 3 + 1, sl]
                    + a2 * rows_v[r * 3 + 2, sl]
                )
                return carry2

            lax.fori_loop(0, MD // 16, cchunk, 0, unroll=4)
            return carry

        lax.fori_loop(0, _RW, combine_row, 0)
        pltpu.sync_copy(out_v, out_hbm.at[pl.ds(base, _RW)])

    return retrieve


def _outa_body(ctrl_ref, woh_ref, bo_ref, out_ref):
    out_ref[...] = (
        lax.dot_general(ctrl_ref[...], woh_ref[...], (((1,), (1,)), ((), ())))
        + bo_ref[...]
    )


def _outa(ctrl, Wo, bo):
    return pl.pallas_call(
        _outa_body,
        grid=(1,),
        in_specs=[
            pl.BlockSpec((N, H), lambda i: (0, 0)),
            pl.BlockSpec((O, H), lambda i: (0, 0)),
            pl.BlockSpec((1, O), lambda i: (0, 0)),
        ],
        out_specs=pl.BlockSpec((N, O), lambda i: (0, 0)),
        out_shape=jax.ShapeDtypeStruct((N, O), jnp.float32),
    )(ctrl, Wo, bo)


def _outb_body(outa_ref, mem_ref, wc_ref, bc_ref, wom_ref, out_ref):
    mem_out = (
        lax.dot_general(mem_ref[...], wc_ref[...], (((1,), (1,)), ((), ())))
        + bc_ref[...]
    )
    res = outa_ref[...] + lax.dot_general(
        mem_out, wom_ref[...], (((1,), (1,)), ((), ()))
    )
    for t in range(S):
        out_ref[:, t, :] = res[t * B:(t + 1) * B, :]


def _outb(outa, mem_raw, Wc, bc, Wo):
    return pl.pallas_call(
        _outb_body,
        grid=(1,),
        in_specs=[
            pl.BlockSpec((N, O), lambda s: (0, 0)),
            pl.BlockSpec((N, MD), lambda s: (0, 0)),
            pl.BlockSpec((MD, MD), lambda s: (0, 0)),
            pl.BlockSpec((1, MD), lambda s: (0, 0)),
            pl.BlockSpec((O, MD), lambda s: (0, H // MD)),
        ],
        out_specs=pl.BlockSpec((B, S, O), lambda s: (0, 0, 0)),
        out_shape=jax.ShapeDtypeStruct((B, S, O), jnp.float32),
    )(outa, mem_raw, Wc, bc, Wo)


def kernel(x, W_ih, W_hh, b_ih, b_hh, Wq, bq, Wa, ba, Wc, bc, Wo, bo,
           mem_keys, mem_values):
    xt = jnp.transpose(x, (1, 0, 2)).reshape(N, I)
    ctrl, sims, va2 = _forward_tc(xt, W_ih, W_hh, b_ih.reshape(1, G4),
                                  b_hh.reshape(1, G4), Wq,
                                  bq.reshape(1, MD), mem_keys, mem_values,
                                  Wa.reshape(1, MD), ba.reshape(1, 1))
    mem_raw = _make_retrieve()(sims, va2, mem_values)
    outa = _outa(ctrl, Wo, bo.reshape(1, O))
    return _outb(outa, mem_raw, Wc, bc.reshape(1, MD), Wo)


# speedup vs baseline: 1.0776x; 1.0438x over previous
import functools

import jax
import jax.numpy as jnp
from jax import lax
from jax.experimental import pallas as pl
from jax.experimental.pallas import tpu as pltpu
from jax.experimental.pallas import tpu_sc as plsc

B, S, I, H, MD, M, O, TOPK = 32, 16, 1024, 1024, 256, 1024, 1024, 3
N = B * S
G4 = 4 * H
NEG = -1e30


def _fwd_body(x_ref, wih_ref, whh_ref, bih_ref, bhh_ref, wq_ref, bq_ref,
              keys_ref, vals_ref, wa_ref, ba_ref, ctrl_ref, sims_ref, va_ref,
              xg_ref, hs_ref, h_ref, c_ref):
    for t in range(S):
        hs_ref[t * B:(t + 1) * B, :] = x_ref[:, t, :]
    xg_ref[...] = (
        lax.dot_general(hs_ref[...], wih_ref[...], (((1,), (1,)), ((), ())))
        + bih_ref[...] + bhh_ref[...]
    )
    h_ref[...] = jnp.zeros_like(h_ref)
    c_ref[...] = jnp.zeros_like(c_ref)

    def step(t, carry):
        g = xg_ref[pl.ds(t * B, B), :] + lax.dot_general(
            h_ref[...], whh_ref[...], (((1,), (1,)), ((), ()))
        )
        i_g = jax.nn.sigmoid(g[:, 0 * H:1 * H])
        f_g = jax.nn.sigmoid(g[:, 1 * H:2 * H])
        g_g = jnp.tanh(g[:, 2 * H:3 * H])
        o_g = jax.nn.sigmoid(g[:, 3 * H:4 * H])
        c = f_g * c_ref[...] + i_g * g_g
        h = o_g * jnp.tanh(c)
        c_ref[...] = c
        h_ref[...] = h
        hs_ref[pl.ds(t * B, B), :] = h
        return carry

    lax.fori_loop(0, S, step, 0)

    ctrl = hs_ref[...]
    ctrl_ref[...] = ctrl
    q = (
        lax.dot_general(ctrl, wq_ref[...], (((1,), (1,)), ((), ())))
        + bq_ref[...]
    )
    qn = q / jnp.maximum(
        jnp.sqrt(jnp.sum(q * q, axis=1, keepdims=True)), 1e-12
    )
    k = keys_ref[...]
    kn = k / jnp.maximum(
        jnp.sqrt(jnp.sum(k * k, axis=1, keepdims=True)), 1e-12
    )
    sims_ref[...] = jnp.reshape(
        lax.dot_general(qn, kn, (((1,), (1,)), ((), ()))), (N * M,)
    )
    va_ref[...] = jnp.sum(vals_ref[...] * wa_ref[...], axis=1) + ba_ref[0, 0]


def _forward_tc(x, W_ih, W_hh, b_ih, b_hh, Wq, bq, mem_keys, mem_values,
                Wa, ba):
    return pl.pallas_call(
        _fwd_body,
        out_shape=(
            jax.ShapeDtypeStruct((N, H), jnp.float32),
            jax.ShapeDtypeStruct((N * M,), jnp.float32),
            jax.ShapeDtypeStruct((M,), jnp.float32),
        ),
        scratch_shapes=[
            pltpu.VMEM((N, G4), jnp.float32),
            pltpu.VMEM((N, H), jnp.float32),
            pltpu.VMEM((B, H), jnp.float32),
            pltpu.VMEM((B, H), jnp.float32),
        ],
    )(x, W_ih, W_hh, b_ih, b_hh, Wq, bq, mem_keys, mem_values, Wa, ba)


_NW = 32
_RW = N // _NW


@functools.lru_cache(maxsize=1)
def _make_retrieve():
    mesh = plsc.VectorSubcoreMesh(core_axis_name="c", subcore_axis_name="s")

    @functools.partial(
        pl.kernel,
        out_type=jax.ShapeDtypeStruct((N, MD), jnp.float32),
        mesh=mesh,
        compiler_params=pltpu.CompilerParams(needs_layout_passes=False),
        scratch_types=[
            pltpu.VMEM((_RW * M,), jnp.float32),
            pltpu.VMEM((M,), jnp.float32),
            pltpu.VMEM((_RW * TOPK,), jnp.int32),
            pltpu.VMEM((_RW * TOPK, MD), jnp.float32),
            pltpu.VMEM((_RW, MD), jnp.float32),
            pltpu.VMEM((TOPK * 16,), jnp.float32),
            pltpu.SemaphoreType.DMA,
        ],
    )
    def retrieve(sims_hbm, va_hbm, vals_hbm, out_hbm,
                 sims_v, va_v, idx_v, rows_v, out_v, att_v, sem):
        wid = lax.axis_index("s") * 2 + lax.axis_index("c")
        base = wid * _RW
        pltpu.sync_copy(sims_hbm.at[pl.ds(base * M, _RW * M)], sims_v)
        pltpu.sync_copy(va_hbm, va_v)

        lane = lax.broadcasted_iota(jnp.int32, (16,), 0)
        zero = jnp.zeros((16,), jnp.int32)
        negs = jnp.full((16,), NEG, jnp.float32)
        big = jnp.full((16,), 2 ** 30, jnp.int32)

        def topk_row(r, carry):
            def chunk(c, st):
                bv1, bv2, bv3, bi1, bi2, bi3, gi = st
                v = sims_v[pl.ds(r * M + c * 16, 16)]
                c1 = v > bv1
                c2 = v > bv2
                c3 = v > bv3
                nb1 = jnp.where(c1, v, bv1)
                ni1 = jnp.where(c1, gi, bi1)
                nb2 = jnp.where(c1, bv1, jnp.where(c2, v, bv2))
                ni2 = jnp.where(c1, bi1, jnp.where(c2, gi, bi2))
                nb3 = jnp.where(c2, bv2, jnp.where(c3, v, bv3))
                ni3 = jnp.where(c2, bi2, jnp.where(c3, gi, bi3))
                return nb1, nb2, nb3, ni1, ni2, ni3, gi + 16

            bv1, bv2, bv3, bi1, bi2, bi3, _ = lax.fori_loop(
                0, M // 16, chunk,
                (negs, negs, negs, zero, zero, zero, lane), unroll=8,
            )

            def extract(st):
                bv1, bv2, bv3, bi1, bi2, bi3 = st
                m = jnp.max(bv1)
                i_t = jnp.min(jnp.where(bv1 == m, bi1, big))
                shift = bi1 == i_t
                bv1 = jnp.where(shift, bv2, bv1)
                bi1 = jnp.where(shift, bi2, bi1)
                bv2 = jnp.where(shift, bv3, bv2)
                bi2 = jnp.where(shift, bi3, bi2)
                return (bv1, bv2, bv3, bi1, bi2, bi3), i_t

            st = (bv1, bv2, bv3, bi1, bi2, bi3)
            st, i0 = extract(st)
            st, i1 = extract(st)
            _, i2 = extract(st)
            vec = jnp.where(lane == 0, zero + i0,
                            jnp.where(lane == 1, zero + i1, zero + i2))
            plsc.store_scatter(idx_v, [r * TOPK + lane], vec,
                               mask=lane < TOPK)
            return carry

        lax.fori_loop(0, _RW, topk_row, 0)

        i0 = plsc.load_gather(idx_v, [lane * TOPK])
        i1 = plsc.load_gather(idx_v, [lane * TOPK + 1])
        i2 = plsc.load_gather(idx_v, [lane * TOPK + 2])
        l0 = plsc.load_gather(va_v, [i0])
        l1 = plsc.load_gather(va_v, [i1])
        l2 = plsc.load_gather(va_v, [i2])
        m = jnp.maximum(jnp.maximum(l0, l1), l2)
        e0 = jnp.exp(l0 - m)
        e1 = jnp.exp(l1 - m)
        e2 = jnp.exp(l2 - m)
        inv = 1.0 / (e0 + e1 + e2)
        att_v[pl.ds(0, 16)] = e0 * inv
        att_v[pl.ds(16, 16)] = e1 * inv
        att_v[pl.ds(32, 16)] = e2 * inv

        pltpu.async_copy(vals_hbm.at[idx_v], rows_v, sem).wait()

        def combine_row(r, carry):
            a0 = plsc.load_gather(att_v, [zero + r])
            a1 = plsc.load_gather(att_v, [zero + (16 + r)])
            a2 = plsc.load_gather(att_v, [zero + (32 + r)])

            def cchunk(j, carry2):
                sl = pl.ds(j * 16, 16)
                out_v[r, sl] = (
                    a0 * rows_v[r * 3 + 0, sl]
                    + a1 * rows_v[r * 3 + 1, sl]
                    + a2 * rows_v[r * 3 + 2, sl]
                )
                return carry2

            lax.fori_loop(0, MD // 16, cchunk, 0, unroll=4)
            return carry

        lax.fori_loop(0, _RW, combine_row, 0)
        pltpu.sync_copy(out_v, out_hbm.at[pl.ds(base, _RW)])

    return retrieve


def _outa_body(ctrl_ref, woh_ref, bo_ref, out_ref):
    out_ref[...] = (
        lax.dot_general(ctrl_ref[...], woh_ref[...], (((1,), (1,)), ((), ())))
        + bo_ref[...]
    )


def _outa(ctrl, Wo, bo):
    return pl.pallas_call(
        _outa_body,
        grid=(1,),
        in_specs=[
            pl.BlockSpec((N, H), lambda i: (0, 0)),
            pl.BlockSpec((O, H), lambda i: (0, 0)),
            pl.BlockSpec((1, O), lambda i: (0, 0)),
        ],
        out_specs=pl.BlockSpec((N, O), lambda i: (0, 0)),
        out_shape=jax.ShapeDtypeStruct((N, O), jnp.float32),
    )(ctrl, Wo, bo)


def _outb_body(outa_ref, mem_ref, wc_ref, bc_ref, wom_ref, out_ref):
    mem_out = (
        lax.dot_general(mem_ref[...], wc_ref[...], (((1,), (1,)), ((), ())))
        + bc_ref[...]
    )
    res = outa_ref[...] + lax.dot_general(
        mem_out, wom_ref[...], (((1,), (1,)), ((), ()))
    )
    for t in range(S):
        out_ref[:, t, :] = res[t * B:(t + 1) * B, :]


def _outb(outa, mem_raw, Wc, bc, Wo):
    return pl.pallas_call(
        _outb_body,
        grid=(1,),
        in_specs=[
            pl.BlockSpec((N, O), lambda s: (0, 0)),
            pl.BlockSpec((N, MD), lambda s: (0, 0)),
            pl.BlockSpec((MD, MD), lambda s: (0, 0)),
            pl.BlockSpec((1, MD), lambda s: (0, 0)),
            pl.BlockSpec((O, MD), lambda s: (0, H // MD)),
        ],
        out_specs=pl.BlockSpec((B, S, O), lambda s: (0, 0, 0)),
        out_shape=jax.ShapeDtypeStruct((B, S, O), jnp.float32),
    )(outa, mem_raw, Wc, bc, Wo)


def kernel(x, W_ih, W_hh, b_ih, b_hh, Wq, bq, Wa, ba, Wc, bc, Wo, bo,
           mem_keys, mem_values):
    ctrl, sims, va2 = _forward_tc(x, W_ih, W_hh, b_ih.reshape(1, G4),
                                  b_hh.reshape(1, G4), Wq,
                                  bq.reshape(1, MD), mem_keys, mem_values,
                                  Wa.reshape(1, MD), ba.reshape(1, 1))
    mem_raw = _make_retrieve()(sims, va2, mem_values)
    outa = _outa(ctrl, Wo, bo.reshape(1, O))
    return _outb(outa, mem_raw, Wc, bc.reshape(1, MD), Wo)
